# baseline (speedup 1.0000x reference)
import functools

import jax
import jax.numpy as jnp
from jax import lax
from jax.experimental import pallas as pl
from jax.experimental.pallas import tpu as pltpu
from jax.experimental.pallas import tpu_sc as plsc

N = 50000
H = 128
M = 200000
CK = 80
NC = M // CK
CPW = 20
WIN = M // (CK * CPW)
HC = 32
NU = 12
NP = 15
NW = 32
S0 = 3200
SL = N - 15 * S0
NB = NC // 2

_mesh = plsc.VectorSubcoreMesh(core_axis_name="c", subcore_axis_name="s")


@functools.partial(
    pl.kernel, mesh=_mesh,
    out_type=jax.ShapeDtypeStruct((3 * M, H), jnp.float32),
    scratch_types=[
        pltpu.VMEM((CPW, CK), jnp.int32),
        pltpu.VMEM((4 * CK, H), jnp.float32),
        pltpu.SemaphoreType.DMA,
    ],
)
def _sc_gather(x_hbm, idx_hbm, out_hbm, idxb, gbuf, sem):
  wid = lax.axis_index("s") * 2 + lax.axis_index("c")
  n_win = 3 * WIN
  n_my = jnp.where(wid < (n_win % NW), n_win // NW + 1, n_win // NW)

  def body(t, carry):
    win = wid + t * NW
    i = win // WIN
    w = win % WIN
    pltpu.sync_copy(idx_hbm.at[i, w], idxb)
    for h in range(CPW // 4):
      cps = [
          pltpu.async_copy(x_hbm.at[idxb.at[h * 4 + j]],
                           gbuf.at[pl.ds(j * CK, CK)], sem)
          for j in range(4)
      ]
      for cp in cps:
        cp.wait()
      pltpu.sync_copy(gbuf, out_hbm.at[pl.ds((win * CPW + h * 4) * CK, 4 * CK)])
    return carry

  lax.fori_loop(0, n_my, body, 0)


@functools.partial(
    pl.kernel, mesh=_mesh,
    out_type=jax.ShapeDtypeStruct((NP, 2, N, HC), jnp.float32),
    scratch_types=[
        pltpu.VMEM_SHARED((N, HC), jnp.float32),
        pltpu.VMEM((CK,), jnp.int32),
        pltpu.VMEM((CK,), jnp.int32),
        pltpu.VMEM((2 * CK, HC), jnp.float32),
        pltpu.VMEM((CK, HC), jnp.float32),
        pltpu.SemaphoreType.DMA,
    ],
)
def _sc_scatter(y_hbm, idx_hbm, z_hbm, out_hbm,
                acc, ix0, ix1, ybuf, onesb, sem):
  cid = lax.axis_index("c")
  sid = lax.axis_index("s")
  wid = sid * 2 + cid
  ixs = [ix0, ix1]
  n_my = jnp.where(wid < (NB % NW), NB // NW + 1, NB // NW)

  def ones_fill(t, carry):
    onesb[t, pl.ds(0, 16)] = jnp.ones((16,), jnp.float32)
    onesb[t, pl.ds(16, 16)] = jnp.ones((16,), jnp.float32)
    return carry

  lax.fori_loop(0, CK, ones_fill, 0)

  def zero_acc():
    @pl.when(sid < 15)
    def _():
      pltpu.sync_copy(z_hbm, acc.at[pl.ds(sid * S0, S0)])

    @pl.when(sid == 15)
    def _():
      pltpu.sync_copy(z_hbm.at[pl.ds(0, SL)], acc.at[pl.ds(15 * S0, SL)])

  def dump_acc(u):
    @pl.when(sid < 15)
    def _():
      s = pl.ds(sid * S0, S0)
      pltpu.sync_copy(acc.at[s], out_hbm.at[u, cid, s, :])

    @pl.when(sid == 15)
    def _():
      s = pl.ds(15 * S0, SL)
      pltpu.sync_copy(acc.at[s], out_hbm.at[u, cid, s, :])

  for u in range(NP):
    i = u // 4 if u < NU else u - NU
    zero_acc()
    plsc.subcore_barrier()

    if u < NU:
      def body(t, carry, u=u, i=i):
        b = wid + t * NW
        cps = [
            pltpu.async_copy(
                idx_hbm.at[pl.ds((i * NC + b * 2 + q) * CK, CK)],
                ixs[q], sem)
            for q in range(2)
        ]
        cps.append(
            pltpu.async_copy(y_hbm.at[u, pl.ds(b * 2 * CK, 2 * CK), :],
                             ybuf, sem))
        for cp in cps:
          cp.wait()
        for q in range(2):
          pltpu.sync_copy(ybuf.at[pl.ds(q * CK, CK)], acc.at[ixs[q]],
                          add=True)
        return carry
    else:
      def body(t, carry, u=u, i=i):
        b = wid + t * NW
        cps = [
            pltpu.async_copy(
                idx_hbm.at[pl.ds((i * NC + b * 2 + q) * CK, CK)],
                ixs[q], sem)
            for q in range(2)
        ]
        for cp in cps:
          cp.wait()
        for q in range(2):
          pltpu.sync_copy(onesb, acc.at[ixs[q]], add=True)
        return carry

    lax.fori_loop(0, n_my, body, 0)
    plsc.subcore_barrier()
    dump_acc(u)
    plsc.subcore_barrier()


@functools.partial(
    pl.kernel, mesh=_mesh,
    out_type=jax.ShapeDtypeStruct((2, N, HC), jnp.float32),
    scratch_types=[
        pltpu.VMEM_SHARED((N, HC), jnp.float32),
        pltpu.SemaphoreType.DMA,
    ],
)
def _sc_probe(z_hbm, out_hbm, acc, sem):
  cid = lax.axis_index("c")
  sid = lax.axis_index("s")

  @pl.when(sid < 15)
  def _():
    pltpu.sync_copy(z_hbm, acc.at[pl.ds(sid * S0, S0)])

  @pl.when(sid == 15)
  def _():
    pltpu.sync_copy(z_hbm.at[pl.ds(0, SL)], acc.at[pl.ds(15 * S0, SL)])

  plsc.subcore_barrier()

  @pl.when(sid < 15)
  def _():
    s = pl.ds(sid * S0, S0)
    pltpu.sync_copy(acc.at[s], out_hbm.at[cid, s, :])

  @pl.when(sid == 15)
  def _():
    s = pl.ds(15 * S0, SL)
    pltpu.sync_copy(acc.at[s], out_hbm.at[cid, s, :])


_BM = 2000


def _tc_matmul_body(g_ref, w_ref, y_ref, st_ref, scr):
  j = pl.program_id(0)
  g = g_ref[...]
  w = w_ref[...]
  y = jnp.dot(g[0], w[0], preferred_element_type=jnp.float32)
  y += jnp.dot(g[1], w[1], preferred_element_type=jnp.float32)
  y += jnp.dot(g[2], w[2], preferred_element_type=jnp.float32)
  for u in range(NU):
    y_ref[u] = y[:, u * HC:(u + 1) * HC]
  s = jnp.stack([jnp.sum(y, axis=0), jnp.sum(y * y, axis=0)])

  @pl.when(j == 0)
  def _():
    scr[...] = jnp.zeros_like(scr)

  scr[...] += s

  @pl.when(j == M // _BM - 1)
  def _():
    st_ref[...] = scr[...][:, None, :]


_R = 1000


def _tc_finalize_body(u_ref, a_ref, b_ref, rec_ref, st_ref, scr):
  j = pl.program_id(0)
  uu = u_ref[...][:, 0] + u_ref[...][:, 1]
  deg_i = [uu[NU + i, :, 0] for i in range(3)]
  parts = []
  for hc in range(4):
    acc = uu[hc] * a_ref[hc, 0][None, :]
    acc += uu[4 + hc] * a_ref[4 + hc, 0][None, :]
    acc += uu[8 + hc] * a_ref[8 + hc, 0][None, :]
    parts.append(acc)
  vs = jnp.concatenate(parts, axis=1)
  vs += (deg_i[0][:, None] * b_ref[0, 0][None, :]
         + deg_i[1][:, None] * b_ref[1, 0][None, :]
         + deg_i[2][:, None] * b_ref[2, 0][None, :])
  deg = deg_i[0] + deg_i[1] + deg_i[2]
  safe = jnp.where(deg > 0, deg, 1.0)
  rec = jnp.where((deg > 0)[:, None], vs / safe[:, None], 0.0)
  rec_ref[...] = rec
  s = jnp.stack([jnp.sum(rec, axis=0), jnp.sum(rec * rec, axis=0)])

  @pl.when(j == 0)
  def _():
    scr[...] = jnp.zeros_like(scr)

  scr[...] += s

  @pl.when(j == N // _R - 1)
  def _():
    st_ref[...] = scr[...][:, None, :]


def _tc_apply_body(rec_ref, a_ref, b_ref, out_ref):
  out_ref[...] = rec_ref[...] * a_ref[...][0] + b_ref[...][0]


def kernel(x, W, gamma1, beta1, gamma2, beta2, cw, clauses):
  idx_t = clauses.T
  idx4 = idx_t.reshape(3, WIN, CPW, CK)
  idxflat = idx_t.reshape(3 * M)
  z32 = jnp.zeros((S0, HC), jnp.float32)

  gb = _sc_gather(x, idx4).reshape(3, M, H)

  w3 = W.reshape(3, H, 3 * H)
  y12, st1 = pl.pallas_call(
      _tc_matmul_body,
      grid=(M // _BM,),
      in_specs=[
          pl.BlockSpec((3, _BM, H), lambda j: (0, j, 0)),
          pl.BlockSpec((3, H, 3 * H), lambda j: (0, 0, 0)),
      ],
      out_specs=[
          pl.BlockSpec((NU, _BM, HC), lambda j: (0, j, 0)),
          pl.BlockSpec((2, 1, 3 * H), lambda j: (0, 0, 0)),
      ],
      out_shape=[
          jax.ShapeDtypeStruct((NU, M, HC), jnp.float32),
          jax.ShapeDtypeStruct((2, 1, 3 * H), jnp.float32),
      ],
      scratch_shapes=[pltpu.VMEM((2, 3 * H), jnp.float32)],
  )(gb, w3)
  st1 = st1.reshape(2, 3 * H)

  mean1 = st1[0] / M
  var1 = st1[1] / M - mean1 * mean1
  rs1 = lax.rsqrt(var1 + 1e-3)
  a1 = cw * gamma1 * rs1
  b1 = cw * (beta1 - mean1 * gamma1 * rs1)

  USE_SC_SCATTER = False
  if USE_SC_SCATTER:
    u_parts = _sc_scatter(y12, idxflat, z32)
  else:
    ups = []
    for u in range(NU):
      z = jnp.zeros((N, HC), jnp.float32)
      ups.append(z.at[clauses[:, u // 4]].add(y12[u]))
    for i in range(3):
      z = jnp.zeros((N, HC), jnp.float32)
      ups.append(z.at[clauses[:, i]].add(jnp.ones((M, HC), jnp.float32)))
    u_parts = jnp.stack(ups)[:, None, :, :] * jnp.array([1.0, 0.0])[None, :, None, None]

  rec_raw, st2 = pl.pallas_call(
      _tc_finalize_body,
      grid=(N // _R,),
      in_specs=[
          pl.BlockSpec((NP, 2, _R, HC), lambda j: (0, 0, j, 0)),
          pl.BlockSpec((NU, 1, HC), lambda j: (0, 0, 0)),
          pl.BlockSpec((3, 1, H), lambda j: (0, 0, 0)),
      ],
      out_specs=[
          pl.BlockSpec((_R, H), lambda j: (j, 0)),
          pl.BlockSpec((2, 1, H), lambda j: (0, 0, 0)),
      ],
      out_shape=[
          jax.ShapeDtypeStruct((N, H), jnp.float32),
          jax.ShapeDtypeStruct((2, 1, H), jnp.float32),
      ],
      scratch_shapes=[pltpu.VMEM((2, H), jnp.float32)],
  )(u_parts, a1.reshape(NU, 1, HC), b1.reshape(3, 1, H))
  st2 = st2.reshape(2, H)

  mean2 = st2[0] / N
  var2 = st2[1] / N - mean2 * mean2
  rs2 = lax.rsqrt(var2 + 1e-3)
  a2 = (gamma2 * rs2).reshape(1, 1, H)
  b2 = (beta2 - mean2 * gamma2 * rs2).reshape(1, 1, H)

  return pl.pallas_call(
      _tc_apply_body,
      grid=(N // _R,),
      in_specs=[
          pl.BlockSpec((_R, H), lambda j: (j, 0)),
          pl.BlockSpec((1, 1, H), lambda j: (0, 0, 0)),
          pl.BlockSpec((1, 1, H), lambda j: (0, 0, 0)),
      ],
      out_specs=pl.BlockSpec((_R, H), lambda j: (j, 0)),
      out_shape=jax.ShapeDtypeStruct((N, H), jnp.float32),
  )(rec_raw, a2, b2)
